# baseline (speedup 1.0000x reference)
import functools

import jax
import jax.numpy as jnp
from jax import lax
from jax.experimental import pallas as pl
from jax.experimental.pallas import tpu as pltpu
from jax.experimental.pallas import tpu_sc as plsc

N = 10000
E = 320000
R = 8
CH = 128
CF = 16
NCHUNK = CH // CF
IDXW = 128
GROUP = 4
RING = 3
E_PAD = 327680
ROWS = E_PAD // IDXW
TILES = 16
ROWS_PER_TILE = ROWS // TILES
STEPS = ROWS_PER_TILE // GROUP
AGG_REAL = R * N
DUMMY = AGG_REAL
AGG_ROWS = 80128
ZROWS = 128
OUT_STRIPE = AGG_REAL // TILES
BN = 2000


def _zero_fill(ref, nrows):
    def body(i, _):
        ref[i, :] = jnp.zeros((CF,), jnp.float32)
        return 0
    lax.fori_loop(0, nrows, body, 0)


XSTRIPE = N // TILES
QSTEPS = 10


def _agg_body(xc, srcr, tgtr, out, agg_sh, src_all, tgt_all, rows_v,
              zer_v, gsem, ssem):
    core = lax.axis_index("c")
    sub = lax.axis_index("s")
    _zero_fill(zer_v, ZROWS)
    for i_c in range(NCHUNK // 2):
        c = core * (NCHUNK // 2) + i_c
        def zbody(z, _):
            pltpu.sync_copy(zer_v,
                            agg_sh.at[pl.ds(sub * 5008 + z * ZROWS, ZROWS)])
            return 0
        lax.fori_loop(0, 39, zbody, 0)
        pltpu.sync_copy(zer_v.at[pl.ds(0, 16)],
                        agg_sh.at[pl.ds(sub * 5008 + 39 * ZROWS, 16)])
        plsc.subcore_barrier()

        xc_c = xc.at[c]

        def fire_gathers(g, buf):
            for j in range(GROUP):
                pltpu.async_copy(
                    xc_c.at[src_all.at[g].at[j]],
                    rows_v.at[pl.ds((buf * GROUP + j) * IDXW, IDXW)], gsem)

        def fire_scatters(g, buf):
            for j in range(GROUP):
                pltpu.async_copy(
                    rows_v.at[pl.ds((buf * GROUP + j) * IDXW, IDXW)],
                    agg_sh.at[tgt_all.at[g].at[j]], ssem, add=True)

        def drain(sem):
            pltpu.make_async_copy(
                xc.at[0].at[pl.ds(0, GROUP * IDXW)],
                rows_v.at[pl.ds(0, GROUP * IDXW)], sem).wait()

        def step(g, _):
            @pl.when(g >= 1)
            def _():
                drain(ssem)
            drain(gsem)
            fire_scatters(g, g % RING)

            @pl.when(g < QSTEPS - RING + 1)
            def _():
                fire_gathers(g + RING - 1, (g + RING - 1) % RING)
            return 0

        for q in range(STEPS // QSTEPS):
            pltpu.sync_copy(srcr.at[sub].at[pl.ds(q * QSTEPS, QSTEPS)], src_all)
            pltpu.sync_copy(tgtr.at[sub].at[pl.ds(q * QSTEPS, QSTEPS)], tgt_all)
            for p in range(RING - 1):
                fire_gathers(p, p)
            lax.fori_loop(0, QSTEPS, step, 0)
            drain(ssem)
        plsc.subcore_barrier()
        r_s = sub // 2
        n0 = (sub % 2) * OUT_STRIPE
        pltpu.sync_copy(
            agg_sh.at[pl.ds(sub * OUT_STRIPE, OUT_STRIPE)],
            out.at[r_s, pl.ds(n0, OUT_STRIPE), pl.ds(c * CF, CF)])
        plsc.subcore_barrier()


_agg_call = pl.kernel(
    _agg_body,
    out_type=jax.ShapeDtypeStruct((R, N, CH), jnp.float32),
    mesh=plsc.VectorSubcoreMesh(core_axis_name="c", subcore_axis_name="s"),
    scratch_types=[
        pltpu.VMEM_SHARED((AGG_ROWS, CF), jnp.float32),
        pltpu.VMEM((QSTEPS, GROUP, IDXW), jnp.int32),
        pltpu.VMEM((QSTEPS, GROUP, IDXW), jnp.int32),
        pltpu.VMEM((RING * GROUP * IDXW, CF), jnp.float32),
        pltpu.VMEM((ZROWS, CF), jnp.float32),
        pltpu.SemaphoreType.DMA,
        pltpu.SemaphoreType.DMA,
    ],
    compiler_params=pltpu.CompilerParams(use_tc_tiling_on_sc=False),
)


def _cnt_body(tgtr, out, cnt_sh, tgt_idx, ones_v, zer_v):
    core = lax.axis_index("c")
    sub = lax.axis_index("s")
    _zero_fill(zer_v, ZROWS)

    def fill(i, _):
        ones_v[i, :] = jnp.ones((CF,), jnp.float32)
        return 0
    lax.fori_loop(0, IDXW, fill, 0)

    def zbody(z, _):
        pltpu.sync_copy(zer_v, cnt_sh.at[pl.ds(sub * 5008 + z * ZROWS, ZROWS)])
        return 0
    lax.fori_loop(0, 39, zbody, 0)
    pltpu.sync_copy(zer_v.at[pl.ds(0, 16)],
                    cnt_sh.at[pl.ds(sub * 5008 + 39 * ZROWS, 16)])
    plsc.subcore_barrier()
    half = ROWS // 2
    per_tile = half // TILES

    def step(g, _):
        base = core * half + sub * per_tile + g * GROUP
        pltpu.sync_copy(tgtr.at[pl.ds(base, GROUP)], tgt_idx)
        for j in range(GROUP):
            pltpu.sync_copy(ones_v, cnt_sh.at[tgt_idx.at[j]], add=True)
        return 0

    lax.fori_loop(0, per_tile // GROUP, step, 0)
    plsc.subcore_barrier()
    pltpu.sync_copy(cnt_sh.at[pl.ds(sub * OUT_STRIPE, OUT_STRIPE)],
                    out.at[core].at[pl.ds(sub * OUT_STRIPE, OUT_STRIPE)])


_cnt_call = pl.kernel(
    _cnt_body,
    out_type=jax.ShapeDtypeStruct((2, AGG_REAL, CF), jnp.float32),
    mesh=plsc.VectorSubcoreMesh(core_axis_name="c", subcore_axis_name="s"),
    scratch_types=[
        pltpu.VMEM_SHARED((AGG_ROWS, CF), jnp.float32),
        pltpu.VMEM((GROUP, IDXW), jnp.int32),
        pltpu.VMEM((IDXW, CF), jnp.float32),
        pltpu.VMEM((ZROWS, CF), jnp.float32),
    ],
    compiler_params=pltpu.CompilerParams(use_tc_tiling_on_sc=False),
)


def _dense_body(relu, x_ref, agg_ref, inv_ref, w_ref, root_ref, b_ref, o_ref):
    acc = jnp.dot(x_ref[...], root_ref[...],
                  preferred_element_type=jnp.float32) + b_ref[...]
    for r in range(R):
        scaled = agg_ref[r] * inv_ref[:, r:r + 1]
        acc = acc + jnp.dot(scaled, w_ref[r], preferred_element_type=jnp.float32)
    if relu:
        acc = jnp.maximum(acc, 0.0)
    o_ref[...] = acc


def _dense_call(x, agg, invT, W, root, b, relu):
    return pl.pallas_call(
        functools.partial(_dense_body, relu),
        grid=(N // BN,),
        in_specs=[
            pl.BlockSpec((BN, CH), lambda i: (i, 0)),
            pl.BlockSpec((R, BN, CH), lambda i: (0, i, 0)),
            pl.BlockSpec((BN, R), lambda i: (i, 0)),
            pl.BlockSpec((R, CH, CH), lambda i: (0, 0, 0)),
            pl.BlockSpec((CH, CH), lambda i: (0, 0)),
            pl.BlockSpec((1, CH), lambda i: (0, 0)),
        ],
        out_specs=pl.BlockSpec((BN, CH), lambda i: (i, 0)),
        out_shape=jax.ShapeDtypeStruct((N, CH), jnp.float32),
    )(x, agg, invT, W, root, b)


def kernel(x, edge_index, edge_type, W1, root1, b1, W2, root2, b2, W3, root3, b3):
    src = edge_index[0]
    dst = edge_index[1]
    tgt = edge_type * N + dst
    pad = E_PAD - E
    srcr = jnp.concatenate([src, jnp.zeros((pad,), jnp.int32)]).reshape(ROWS, IDXW)
    tgtr = jnp.concatenate([tgt, jnp.full((pad,), DUMMY, jnp.int32)]).reshape(ROWS, IDXW)
    srcr4 = srcr.reshape(TILES, STEPS, GROUP, IDXW)
    tgtr4 = tgtr.reshape(TILES, STEPS, GROUP, IDXW)

    cnt2 = _cnt_call(tgtr)
    cnt = cnt2[0, :, 0] + cnt2[1, :, 0]
    invT = (1.0 / jnp.maximum(cnt, 1.0)).reshape(R, N).T

    def layer(h, W, root, b, relu):
        hc = h.reshape(N, NCHUNK, CF).transpose(1, 0, 2)
        agg = _agg_call(hc, srcr4, tgtr4)
        return _dense_call(h, agg, invT, W, root, b.reshape(1, CH), relu)

    h1 = layer(x, W1, root1, b1, True)
    h2 = layer(h1, W2, root2, b2, True)
    return layer(h2, W3, root3, b3, False)


# speedup vs baseline: 1.5339x; 1.5339x over previous
import functools

import jax
import jax.numpy as jnp
from jax import lax
from jax.experimental import pallas as pl
from jax.experimental.pallas import tpu as pltpu
from jax.experimental.pallas import tpu_sc as plsc

N = 10000
E = 320000
R = 8
CH = 128
CF = 16
NCHUNK = CH // CF
IDXW = 128
GROUP = 2
RING = 6
E_PAD = 327680
ROWS = E_PAD // IDXW
TILES = 16
ROWS_PER_TILE = ROWS // TILES
STEPS = ROWS_PER_TILE // GROUP
AGG_REAL = R * N
DUMMY = AGG_REAL
AGG_ROWS = 80128
ZROWS = 128
OUT_STRIPE = AGG_REAL // TILES
BN = 2000


def _zero_fill(ref, nrows):
    def body(i, _):
        ref[i, :] = jnp.zeros((CF,), jnp.float32)
        return 0
    lax.fori_loop(0, nrows, body, 0)


XSTRIPE = N // TILES
QSTEPS = 20


def _agg_body(xv, srcr, tgtr, out, agg_sh, xc_sh, src_all, tgt_all, rows_v,
              zer_v, gsem, ssem):
    core = lax.axis_index("c")
    sub = lax.axis_index("s")
    _zero_fill(zer_v, ZROWS)
    for i_c in range(NCHUNK // 2):
        c = core * (NCHUNK // 2) + i_c
        pltpu.sync_copy(
            xv.at[pl.ds(sub * XSTRIPE, XSTRIPE), pl.ds(c * CF, CF)],
            xc_sh.at[pl.ds(sub * XSTRIPE, XSTRIPE)])

        def zbody(z, _):
            pltpu.sync_copy(zer_v,
                            agg_sh.at[pl.ds(sub * 5008 + z * ZROWS, ZROWS)])
            return 0
        lax.fori_loop(0, 39, zbody, 0)
        pltpu.sync_copy(zer_v.at[pl.ds(0, 16)],
                        agg_sh.at[pl.ds(sub * 5008 + 39 * ZROWS, 16)])
        plsc.subcore_barrier()

        def fire_gathers(g, buf):
            for j in range(GROUP):
                pltpu.async_copy(
                    xc_sh.at[src_all.at[g].at[j]],
                    rows_v.at[pl.ds((buf * GROUP + j) * IDXW, IDXW)], gsem)

        def fire_scatters(g, buf):
            for j in range(GROUP):
                pltpu.async_copy(
                    rows_v.at[pl.ds((buf * GROUP + j) * IDXW, IDXW)],
                    agg_sh.at[tgt_all.at[g].at[j]], ssem, add=True)

        def drain(sem):
            pltpu.make_async_copy(
                xv.at[pl.ds(0, GROUP * IDXW), pl.ds(0, CF)],
                rows_v.at[pl.ds(0, GROUP * IDXW)], sem).wait()

        def step(g, _):
            @pl.when(g >= 1)
            def _():
                drain(ssem)
            drain(gsem)
            fire_scatters(g, g % RING)

            @pl.when(g < QSTEPS - RING + 1)
            def _():
                fire_gathers(g + RING - 1, (g + RING - 1) % RING)
            return 0

        for q in range(STEPS // QSTEPS):
            pltpu.sync_copy(srcr.at[sub].at[pl.ds(q * QSTEPS, QSTEPS)], src_all)
            pltpu.sync_copy(tgtr.at[sub].at[pl.ds(q * QSTEPS, QSTEPS)], tgt_all)
            for p in range(RING - 1):
                fire_gathers(p, p)
            lax.fori_loop(0, QSTEPS, step, 0)
            drain(ssem)
        plsc.subcore_barrier()
        r_s = sub // 2
        n0 = (sub % 2) * OUT_STRIPE
        pltpu.sync_copy(
            agg_sh.at[pl.ds(sub * OUT_STRIPE, OUT_STRIPE)],
            out.at[r_s, pl.ds(n0, OUT_STRIPE), pl.ds(c * CF, CF)])
        plsc.subcore_barrier()


_agg_call = pl.kernel(
    _agg_body,
    out_type=jax.ShapeDtypeStruct((R, N, CH), jnp.float32),
    mesh=plsc.VectorSubcoreMesh(core_axis_name="c", subcore_axis_name="s"),
    scratch_types=[
        pltpu.VMEM_SHARED((AGG_ROWS, CF), jnp.float32),
        pltpu.VMEM_SHARED((N, CF), jnp.float32),
        pltpu.VMEM((QSTEPS, GROUP, IDXW), jnp.int32),
        pltpu.VMEM((QSTEPS, GROUP, IDXW), jnp.int32),
        pltpu.VMEM((RING * GROUP * IDXW, CF), jnp.float32),
        pltpu.VMEM((ZROWS, CF), jnp.float32),
        pltpu.SemaphoreType.DMA,
        pltpu.SemaphoreType.DMA,
    ],
    compiler_params=pltpu.CompilerParams(use_tc_tiling_on_sc=False),
)


def _cnt_body(tgtr, out, cnt_sh, tgt_idx, ones_v, zer_v):
    core = lax.axis_index("c")
    sub = lax.axis_index("s")
    _zero_fill(zer_v, ZROWS)

    def fill(i, _):
        ones_v[i, :] = jnp.ones((CF,), jnp.float32)
        return 0
    lax.fori_loop(0, IDXW, fill, 0)

    def zbody(z, _):
        pltpu.sync_copy(zer_v, cnt_sh.at[pl.ds(sub * 5008 + z * ZROWS, ZROWS)])
        return 0
    lax.fori_loop(0, 39, zbody, 0)
    pltpu.sync_copy(zer_v.at[pl.ds(0, 16)],
                    cnt_sh.at[pl.ds(sub * 5008 + 39 * ZROWS, 16)])
    plsc.subcore_barrier()
    half = ROWS // 2
    per_tile = half // TILES

    def step(g, _):
        base = core * half + sub * per_tile + g * GROUP
        pltpu.sync_copy(tgtr.at[pl.ds(base, GROUP)], tgt_idx)
        for j in range(GROUP):
            pltpu.sync_copy(ones_v, cnt_sh.at[tgt_idx.at[j]], add=True)
        return 0

    lax.fori_loop(0, per_tile // GROUP, step, 0)
    plsc.subcore_barrier()
    pltpu.sync_copy(cnt_sh.at[pl.ds(sub * OUT_STRIPE, OUT_STRIPE)],
                    out.at[core].at[pl.ds(sub * OUT_STRIPE, OUT_STRIPE)])


_cnt_call = pl.kernel(
    _cnt_body,
    out_type=jax.ShapeDtypeStruct((2, AGG_REAL, CF), jnp.float32),
    mesh=plsc.VectorSubcoreMesh(core_axis_name="c", subcore_axis_name="s"),
    scratch_types=[
        pltpu.VMEM_SHARED((AGG_ROWS, CF), jnp.float32),
        pltpu.VMEM((GROUP, IDXW), jnp.int32),
        pltpu.VMEM((IDXW, CF), jnp.float32),
        pltpu.VMEM((ZROWS, CF), jnp.float32),
    ],
    compiler_params=pltpu.CompilerParams(use_tc_tiling_on_sc=False),
)


def _dense_body(relu, x_ref, agg_ref, inv_ref, w_ref, root_ref, b_ref, o_ref):
    acc = jnp.dot(x_ref[...], root_ref[...],
                  preferred_element_type=jnp.float32) + b_ref[...]
    for r in range(R):
        scaled = agg_ref[r] * inv_ref[:, r:r + 1]
        acc = acc + jnp.dot(scaled, w_ref[r], preferred_element_type=jnp.float32)
    if relu:
        acc = jnp.maximum(acc, 0.0)
    o_ref[...] = acc


def _dense_call(x, agg, invT, W, root, b, relu):
    return pl.pallas_call(
        functools.partial(_dense_body, relu),
        grid=(N // BN,),
        in_specs=[
            pl.BlockSpec((BN, CH), lambda i: (i, 0)),
            pl.BlockSpec((R, BN, CH), lambda i: (0, i, 0)),
            pl.BlockSpec((BN, R), lambda i: (i, 0)),
            pl.BlockSpec((R, CH, CH), lambda i: (0, 0, 0)),
            pl.BlockSpec((CH, CH), lambda i: (0, 0)),
            pl.BlockSpec((1, CH), lambda i: (0, 0)),
        ],
        out_specs=pl.BlockSpec((BN, CH), lambda i: (i, 0)),
        out_shape=jax.ShapeDtypeStruct((N, CH), jnp.float32),
    )(x, agg, invT, W, root, b)


def kernel(x, edge_index, edge_type, W1, root1, b1, W2, root2, b2, W3, root3, b3):
    src = edge_index[0]
    dst = edge_index[1]
    tgt = edge_type * N + dst
    pad = E_PAD - E
    srcr = jnp.concatenate([src, jnp.zeros((pad,), jnp.int32)]).reshape(ROWS, IDXW)
    tgtr = jnp.concatenate([tgt, jnp.full((pad,), DUMMY, jnp.int32)]).reshape(ROWS, IDXW)
    srcr4 = srcr.reshape(TILES, STEPS, GROUP, IDXW)
    tgtr4 = tgtr.reshape(TILES, STEPS, GROUP, IDXW)

    cnt2 = _cnt_call(tgtr)
    cnt = cnt2[0, :, 0] + cnt2[1, :, 0]
    invT = (1.0 / jnp.maximum(cnt, 1.0)).reshape(R, N).T

    def layer(h, W, root, b, relu):
        agg = _agg_call(h, srcr4, tgtr4)
        return _dense_call(h, agg, invT, W, root, b.reshape(1, CH), relu)

    h1 = layer(x, W1, root1, b1, True)
    h2 = layer(h1, W2, root2, b2, True)
    return layer(h2, W3, root3, b3, False)


# speedup vs baseline: 1.5595x; 1.0167x over previous
import functools

import jax
import jax.numpy as jnp
from jax import lax
from jax.experimental import pallas as pl
from jax.experimental.pallas import tpu as pltpu
from jax.experimental.pallas import tpu_sc as plsc

N = 10000
E = 320000
R = 8
CH = 128
CF = 16
NCHUNK = CH // CF
IDXW = 128
GROUP = 2
RING = 6
E_PAD = 327680
ROWS = E_PAD // IDXW
TILES = 16
ROWS_PER_TILE = ROWS // TILES
STEPS = ROWS_PER_TILE // GROUP
AGG_REAL = R * N
DUMMY = AGG_REAL
AGG_ROWS = 80128
ZROWS = 128
OUT_STRIPE = AGG_REAL // TILES
BN = 2000


def _zero_fill(ref, nrows):
    def body(i, _):
        ref[i, :] = jnp.zeros((CF,), jnp.float32)
        return 0
    lax.fori_loop(0, nrows, body, 0)


XSTRIPE = N // TILES
QSTEPS = 20


def _agg_body(xv, srcr, tgtr, out, agg_sh, xc_sh, src_all, tgt_all, rows_v,
              zer_v, gsem, ssem):
    core = lax.axis_index("c")
    sub = lax.axis_index("s")
    _zero_fill(zer_v, ZROWS)
    for i_c in range(NCHUNK // 2):
        c = core * (NCHUNK // 2) + i_c
        pltpu.sync_copy(
            xv.at[pl.ds(sub * XSTRIPE, XSTRIPE), pl.ds(c * CF, CF)],
            xc_sh.at[pl.ds(sub * XSTRIPE, XSTRIPE)])

        def fire_gathers(g, buf):
            for j in range(GROUP):
                pltpu.async_copy(
                    xc_sh.at[src_all.at[g].at[j]],
                    rows_v.at[pl.ds((buf * GROUP + j) * IDXW, IDXW)], gsem)

        def fire_scatters(g, buf):
            for j in range(GROUP):
                pltpu.async_copy(
                    rows_v.at[pl.ds((buf * GROUP + j) * IDXW, IDXW)],
                    agg_sh.at[tgt_all.at[g].at[j]], ssem, add=True)

        def drain(sem):
            pltpu.make_async_copy(
                xv.at[pl.ds(0, GROUP * IDXW), pl.ds(0, CF)],
                rows_v.at[pl.ds(0, GROUP * IDXW)], sem).wait()

        def step(g, _):
            @pl.when(g >= 1)
            def _():
                drain(ssem)
            drain(gsem)
            fire_scatters(g, g % RING)

            @pl.when(g < QSTEPS - RING + 1)
            def _():
                fire_gathers(g + RING - 1, (g + RING - 1) % RING)
            return 0

        pltpu.sync_copy(srcr.at[sub].at[pl.ds(0, QSTEPS)], src_all)
        pltpu.sync_copy(tgtr.at[sub].at[pl.ds(0, QSTEPS)], tgt_all)
        plsc.subcore_barrier()
        for p in range(RING - 1):
            fire_gathers(p, p)

        def zbody(z, _):
            pltpu.sync_copy(zer_v,
                            agg_sh.at[pl.ds(sub * 5008 + z * ZROWS, ZROWS)])
            return 0
        lax.fori_loop(0, 39, zbody, 0)
        pltpu.sync_copy(zer_v.at[pl.ds(0, 16)],
                        agg_sh.at[pl.ds(sub * 5008 + 39 * ZROWS, 16)])
        plsc.subcore_barrier()

        for q in range(STEPS // QSTEPS):
            if q > 0:
                pltpu.sync_copy(srcr.at[sub].at[pl.ds(q * QSTEPS, QSTEPS)],
                                src_all)
                pltpu.sync_copy(tgtr.at[sub].at[pl.ds(q * QSTEPS, QSTEPS)],
                                tgt_all)
                for p in range(RING - 1):
                    fire_gathers(p, p)
            lax.fori_loop(0, QSTEPS, step, 0)
            drain(ssem)
        plsc.subcore_barrier()
        r_s = sub // 2
        n0 = (sub % 2) * OUT_STRIPE
        pltpu.sync_copy(
            agg_sh.at[pl.ds(sub * OUT_STRIPE, OUT_STRIPE)],
            out.at[r_s, pl.ds(n0, OUT_STRIPE), pl.ds(c * CF, CF)])
        plsc.subcore_barrier()


_agg_call = pl.kernel(
    _agg_body,
    out_type=jax.ShapeDtypeStruct((R, N, CH), jnp.float32),
    mesh=plsc.VectorSubcoreMesh(core_axis_name="c", subcore_axis_name="s"),
    scratch_types=[
        pltpu.VMEM_SHARED((AGG_ROWS, CF), jnp.float32),
        pltpu.VMEM_SHARED((N, CF), jnp.float32),
        pltpu.VMEM((QSTEPS, GROUP, IDXW), jnp.int32),
        pltpu.VMEM((QSTEPS, GROUP, IDXW), jnp.int32),
        pltpu.VMEM((RING * GROUP * IDXW, CF), jnp.float32),
        pltpu.VMEM((ZROWS, CF), jnp.float32),
        pltpu.SemaphoreType.DMA,
        pltpu.SemaphoreType.DMA,
    ],
    compiler_params=pltpu.CompilerParams(use_tc_tiling_on_sc=False),
)


CG = 8


def _cnt_body(tgtr, out, cnt_sh, tgt_idx, ones_v, zer_v, ssem):
    core = lax.axis_index("c")
    sub = lax.axis_index("s")
    _zero_fill(zer_v, ZROWS)

    def fill(i, _):
        ones_v[i, :] = jnp.ones((CF,), jnp.float32)
        return 0
    lax.fori_loop(0, IDXW, fill, 0)

    def zbody(z, _):
        pltpu.sync_copy(zer_v, cnt_sh.at[pl.ds(sub * 5008 + z * ZROWS, ZROWS)])
        return 0
    lax.fori_loop(0, 39, zbody, 0)
    pltpu.sync_copy(zer_v.at[pl.ds(0, 16)],
                    cnt_sh.at[pl.ds(sub * 5008 + 39 * ZROWS, 16)])
    plsc.subcore_barrier()
    half = ROWS // 2
    per_tile = half // TILES

    def step(g, _):
        base = core * half + sub * per_tile + g * CG
        pltpu.sync_copy(tgtr.at[pl.ds(base, CG)], tgt_idx)
        scs = []
        for j in range(CG):
            scs.append(pltpu.async_copy(
                ones_v, cnt_sh.at[tgt_idx.at[j]], ssem, add=True))
        for s in scs:
            s.wait()
        return 0

    lax.fori_loop(0, per_tile // CG, step, 0)
    plsc.subcore_barrier()
    pltpu.sync_copy(cnt_sh.at[pl.ds(sub * OUT_STRIPE, OUT_STRIPE)],
                    out.at[core].at[pl.ds(sub * OUT_STRIPE, OUT_STRIPE)])


_cnt_call = pl.kernel(
    _cnt_body,
    out_type=jax.ShapeDtypeStruct((2, AGG_REAL, CF), jnp.float32),
    mesh=plsc.VectorSubcoreMesh(core_axis_name="c", subcore_axis_name="s"),
    scratch_types=[
        pltpu.VMEM_SHARED((AGG_ROWS, CF), jnp.float32),
        pltpu.VMEM((CG, IDXW), jnp.int32),
        pltpu.VMEM((IDXW, CF), jnp.float32),
        pltpu.VMEM((ZROWS, CF), jnp.float32),
        pltpu.SemaphoreType.DMA,
    ],
    compiler_params=pltpu.CompilerParams(use_tc_tiling_on_sc=False),
)


def _dense_body(relu, x_ref, agg_ref, inv_ref, w_ref, root_ref, b_ref, o_ref):
    acc = jnp.dot(x_ref[...], root_ref[...],
                  preferred_element_type=jnp.float32) + b_ref[...]
    for r in range(R):
        scaled = agg_ref[r] * inv_ref[:, r:r + 1]
        acc = acc + jnp.dot(scaled, w_ref[r], preferred_element_type=jnp.float32)
    if relu:
        acc = jnp.maximum(acc, 0.0)
    o_ref[...] = acc


def _dense_call(x, agg, invT, W, root, b, relu):
    return pl.pallas_call(
        functools.partial(_dense_body, relu),
        grid=(N // BN,),
        in_specs=[
            pl.BlockSpec((BN, CH), lambda i: (i, 0)),
            pl.BlockSpec((R, BN, CH), lambda i: (0, i, 0)),
            pl.BlockSpec((BN, R), lambda i: (i, 0)),
            pl.BlockSpec((R, CH, CH), lambda i: (0, 0, 0)),
            pl.BlockSpec((CH, CH), lambda i: (0, 0)),
            pl.BlockSpec((1, CH), lambda i: (0, 0)),
        ],
        out_specs=pl.BlockSpec((BN, CH), lambda i: (i, 0)),
        out_shape=jax.ShapeDtypeStruct((N, CH), jnp.float32),
    )(x, agg, invT, W, root, b)


def kernel(x, edge_index, edge_type, W1, root1, b1, W2, root2, b2, W3, root3, b3):
    src = edge_index[0]
    dst = edge_index[1]
    tgt = edge_type * N + dst
    pad = E_PAD - E
    srcr = jnp.concatenate([src, jnp.zeros((pad,), jnp.int32)]).reshape(ROWS, IDXW)
    tgtr = jnp.concatenate([tgt, jnp.full((pad,), DUMMY, jnp.int32)]).reshape(ROWS, IDXW)
    srcr4 = srcr.reshape(TILES, STEPS, GROUP, IDXW)
    tgtr4 = tgtr.reshape(TILES, STEPS, GROUP, IDXW)

    cnt2 = _cnt_call(tgtr)
    cnt = cnt2[0, :, 0] + cnt2[1, :, 0]
    invT = (1.0 / jnp.maximum(cnt, 1.0)).reshape(R, N).T

    def layer(h, W, root, b, relu):
        agg = _agg_call(h, srcr4, tgtr4)
        return _dense_call(h, agg, invT, W, root, b.reshape(1, CH), relu)

    h1 = layer(x, W1, root1, b1, True)
    h2 = layer(h1, W2, root2, b2, True)
    return layer(h2, W3, root3, b3, False)


# speedup vs baseline: 1.5920x; 1.0208x over previous
import functools

import jax
import jax.numpy as jnp
from jax import lax
from jax.experimental import pallas as pl
from jax.experimental.pallas import tpu as pltpu
from jax.experimental.pallas import tpu_sc as plsc

N = 10000
E = 320000
R = 8
CH = 128
CF = 16
NCHUNK = CH // CF
IDXW = 128
GROUP = 2
RING = 4
E_PAD = 327680
ROWS = E_PAD // IDXW
TILES = 16
ROWS_PER_TILE = ROWS // TILES
STEPS = ROWS_PER_TILE // GROUP
AGG_REAL = R * N
DUMMY = AGG_REAL
AGG_ROWS = 80128
ZROWS = 128
OUT_STRIPE = AGG_REAL // TILES
BN = 2000


def _zero_fill(ref, nrows):
    def body(i, _):
        ref[i, :] = jnp.zeros((CF,), jnp.float32)
        return 0
    lax.fori_loop(0, nrows, body, 0)


XSTRIPE = N // TILES
QSTEPS = 40


def _agg_body(xv, srcr, tgtr, out, agg_sh, xc_sh, src_all, tgt_all, rows_v,
              zer_v, gsem, ssem):
    core = lax.axis_index("c")
    sub = lax.axis_index("s")
    _zero_fill(zer_v, ZROWS)
    for i_c in range(NCHUNK // 2):
        c = core * (NCHUNK // 2) + i_c
        pltpu.sync_copy(
            xv.at[pl.ds(sub * XSTRIPE, XSTRIPE), pl.ds(c * CF, CF)],
            xc_sh.at[pl.ds(sub * XSTRIPE, XSTRIPE)])

        def fire_gathers(g, buf):
            for j in range(GROUP):
                pltpu.async_copy(
                    xc_sh.at[src_all.at[g].at[j]],
                    rows_v.at[pl.ds((buf * GROUP + j) * IDXW, IDXW)], gsem)

        def fire_scatters(g, buf):
            for j in range(GROUP):
                pltpu.async_copy(
                    rows_v.at[pl.ds((buf * GROUP + j) * IDXW, IDXW)],
                    agg_sh.at[tgt_all.at[g].at[j]], ssem, add=True)

        def drain(sem):
            pltpu.make_async_copy(
                xv.at[pl.ds(0, GROUP * IDXW), pl.ds(0, CF)],
                rows_v.at[pl.ds(0, GROUP * IDXW)], sem).wait()

        def step(g, _):
            @pl.when(g >= 1)
            def _():
                drain(ssem)
            drain(gsem)
            fire_scatters(g, g % RING)

            @pl.when(g < QSTEPS - RING + 1)
            def _():
                fire_gathers(g + RING - 1, (g + RING - 1) % RING)
            return 0

        pltpu.sync_copy(srcr.at[sub].at[pl.ds(0, QSTEPS)], src_all)
        pltpu.sync_copy(tgtr.at[sub].at[pl.ds(0, QSTEPS)], tgt_all)
        plsc.subcore_barrier()
        for p in range(RING - 1):
            fire_gathers(p, p)

        def zbody(z, _):
            pltpu.sync_copy(zer_v,
                            agg_sh.at[pl.ds(sub * 5008 + z * ZROWS, ZROWS)])
            return 0
        lax.fori_loop(0, 39, zbody, 0)
        pltpu.sync_copy(zer_v.at[pl.ds(0, 16)],
                        agg_sh.at[pl.ds(sub * 5008 + 39 * ZROWS, 16)])
        plsc.subcore_barrier()

        for q in range(STEPS // QSTEPS):
            if q > 0:
                pltpu.sync_copy(srcr.at[sub].at[pl.ds(q * QSTEPS, QSTEPS)],
                                src_all)
                pltpu.sync_copy(tgtr.at[sub].at[pl.ds(q * QSTEPS, QSTEPS)],
                                tgt_all)
                for p in range(RING - 1):
                    fire_gathers(p, p)
            lax.fori_loop(0, QSTEPS, step, 0)
            drain(ssem)
        plsc.subcore_barrier()
        r_s = sub // 2
        n0 = (sub % 2) * OUT_STRIPE
        pltpu.sync_copy(
            agg_sh.at[pl.ds(sub * OUT_STRIPE, OUT_STRIPE)],
            out.at[r_s, pl.ds(n0, OUT_STRIPE), pl.ds(c * CF, CF)])
        plsc.subcore_barrier()


_agg_call = pl.kernel(
    _agg_body,
    out_type=jax.ShapeDtypeStruct((R, N, CH), jnp.float32),
    mesh=plsc.VectorSubcoreMesh(core_axis_name="c", subcore_axis_name="s"),
    scratch_types=[
        pltpu.VMEM_SHARED((AGG_ROWS, CF), jnp.float32),
        pltpu.VMEM_SHARED((N, CF), jnp.float32),
        pltpu.VMEM((QSTEPS, GROUP, IDXW), jnp.int32),
        pltpu.VMEM((QSTEPS, GROUP, IDXW), jnp.int32),
        pltpu.VMEM((RING * GROUP * IDXW, CF), jnp.float32),
        pltpu.VMEM((ZROWS, CF), jnp.float32),
        pltpu.SemaphoreType.DMA,
        pltpu.SemaphoreType.DMA,
    ],
    compiler_params=pltpu.CompilerParams(use_tc_tiling_on_sc=False),
)


CG = 8


def _cnt_body(tgtr, out, cnt_sh, tgt_idx, ones_v, zer_v, ssem):
    core = lax.axis_index("c")
    sub = lax.axis_index("s")
    _zero_fill(zer_v, ZROWS)

    def fill(i, _):
        ones_v[i, :] = jnp.ones((CF,), jnp.float32)
        return 0
    lax.fori_loop(0, IDXW, fill, 0)

    def zbody(z, _):
        pltpu.sync_copy(zer_v, cnt_sh.at[pl.ds(sub * 5008 + z * ZROWS, ZROWS)])
        return 0
    lax.fori_loop(0, 39, zbody, 0)
    pltpu.sync_copy(zer_v.at[pl.ds(0, 16)],
                    cnt_sh.at[pl.ds(sub * 5008 + 39 * ZROWS, 16)])
    plsc.subcore_barrier()
    half = ROWS // 2
    per_tile = half // TILES

    def step(g, _):
        base = core * half + sub * per_tile + g * CG
        pltpu.sync_copy(tgtr.at[pl.ds(base, CG)], tgt_idx)
        scs = []
        for j in range(CG):
            scs.append(pltpu.async_copy(
                ones_v, cnt_sh.at[tgt_idx.at[j]], ssem, add=True))
        for s in scs:
            s.wait()
        return 0

    lax.fori_loop(0, per_tile // CG, step, 0)
    plsc.subcore_barrier()
    pltpu.sync_copy(cnt_sh.at[pl.ds(sub * OUT_STRIPE, OUT_STRIPE)],
                    out.at[core].at[pl.ds(sub * OUT_STRIPE, OUT_STRIPE)])


_cnt_call = pl.kernel(
    _cnt_body,
    out_type=jax.ShapeDtypeStruct((2, AGG_REAL, CF), jnp.float32),
    mesh=plsc.VectorSubcoreMesh(core_axis_name="c", subcore_axis_name="s"),
    scratch_types=[
        pltpu.VMEM_SHARED((AGG_ROWS, CF), jnp.float32),
        pltpu.VMEM((CG, IDXW), jnp.int32),
        pltpu.VMEM((IDXW, CF), jnp.float32),
        pltpu.VMEM((ZROWS, CF), jnp.float32),
        pltpu.SemaphoreType.DMA,
    ],
    compiler_params=pltpu.CompilerParams(use_tc_tiling_on_sc=False),
)


def _dense_body(relu, x_ref, agg_ref, inv_ref, w_ref, root_ref, b_ref, o_ref):
    acc = jnp.dot(x_ref[...], root_ref[...],
                  preferred_element_type=jnp.float32) + b_ref[...]
    for r in range(R):
        scaled = agg_ref[r] * inv_ref[:, r:r + 1]
        acc = acc + jnp.dot(scaled, w_ref[r], preferred_element_type=jnp.float32)
    if relu:
        acc = jnp.maximum(acc, 0.0)
    o_ref[...] = acc


def _dense_call(x, agg, invT, W, root, b, relu):
    return pl.pallas_call(
        functools.partial(_dense_body, relu),
        grid=(N // BN,),
        in_specs=[
            pl.BlockSpec((BN, CH), lambda i: (i, 0)),
            pl.BlockSpec((R, BN, CH), lambda i: (0, i, 0)),
            pl.BlockSpec((BN, R), lambda i: (i, 0)),
            pl.BlockSpec((R, CH, CH), lambda i: (0, 0, 0)),
            pl.BlockSpec((CH, CH), lambda i: (0, 0)),
            pl.BlockSpec((1, CH), lambda i: (0, 0)),
        ],
        out_specs=pl.BlockSpec((BN, CH), lambda i: (i, 0)),
        out_shape=jax.ShapeDtypeStruct((N, CH), jnp.float32),
    )(x, agg, invT, W, root, b)


def kernel(x, edge_index, edge_type, W1, root1, b1, W2, root2, b2, W3, root3, b3):
    src = edge_index[0]
    dst = edge_index[1]
    tgt = edge_type * N + dst
    pad = E_PAD - E
    srcr = jnp.concatenate([src, jnp.zeros((pad,), jnp.int32)]).reshape(ROWS, IDXW)
    tgtr = jnp.concatenate([tgt, jnp.full((pad,), DUMMY, jnp.int32)]).reshape(ROWS, IDXW)
    srcr4 = srcr.reshape(TILES, STEPS, GROUP, IDXW)
    tgtr4 = tgtr.reshape(TILES, STEPS, GROUP, IDXW)

    cnt2 = _cnt_call(tgtr)
    cnt = cnt2[0, :, 0] + cnt2[1, :, 0]
    invT = (1.0 / jnp.maximum(cnt, 1.0)).reshape(R, N).T

    def layer(h, W, root, b, relu):
        agg = _agg_call(h, srcr4, tgtr4)
        return _dense_call(h, agg, invT, W, root, b.reshape(1, CH), relu)

    h1 = layer(x, W1, root1, b1, True)
    h2 = layer(h1, W2, root2, b2, True)
    return layer(h2, W3, root3, b3, False)


# speedup vs baseline: 1.6638x; 1.0451x over previous
import functools

import jax
import jax.numpy as jnp
from jax import lax
from jax.experimental import pallas as pl
from jax.experimental.pallas import tpu as pltpu
from jax.experimental.pallas import tpu_sc as plsc

N = 10000
E = 320000
R = 8
CH = 128
CF = 16
NCHUNK = CH // CF
IDXW = 128
GROUP = 2
RING = 4
E_PAD = 327680
ROWS = E_PAD // IDXW
TILES = 16
ROWS_PER_TILE = ROWS // TILES
STEPS = ROWS_PER_TILE // GROUP
AGG_REAL = R * N
DUMMY = AGG_REAL
AGG_ROWS = 80128
ZROWS = 128
OUT_STRIPE = AGG_REAL // TILES
BN = 2000


def _zero_fill(ref, nrows):
    def body(i, _):
        ref[i, :] = jnp.zeros((CF,), jnp.float32)
        return 0
    lax.fori_loop(0, nrows, body, 0)


XSTRIPE = N // TILES
QSTEPS = 40


def _agg_body(xv, srcr, tgtr, out, agg_sh, xc_sh, src_all, tgt_all, rows_v,
              zer_v, gsem, ssem):
    core = lax.axis_index("c")
    sub = lax.axis_index("s")
    _zero_fill(zer_v, ZROWS)
    for i_c in range(NCHUNK // 2):
        c = core * (NCHUNK // 2) + i_c
        st = pltpu.async_copy(
            xv.at[pl.ds(sub * XSTRIPE, XSTRIPE), pl.ds(c * CF, CF)],
            xc_sh.at[pl.ds(sub * XSTRIPE, XSTRIPE)], gsem)

        def fire_gathers(g, buf):
            for j in range(GROUP):
                pltpu.async_copy(
                    xc_sh.at[src_all.at[g].at[j]],
                    rows_v.at[pl.ds((buf * GROUP + j) * IDXW, IDXW)], gsem)

        def fire_scatters(g, buf):
            for j in range(GROUP):
                pltpu.async_copy(
                    rows_v.at[pl.ds((buf * GROUP + j) * IDXW, IDXW)],
                    agg_sh.at[tgt_all.at[g].at[j]], ssem, add=True)

        def drain(sem):
            pltpu.make_async_copy(
                xv.at[pl.ds(0, GROUP * IDXW), pl.ds(0, CF)],
                rows_v.at[pl.ds(0, GROUP * IDXW)], sem).wait()

        def step(g, _):
            @pl.when(g >= 1)
            def _():
                drain(ssem)
            drain(gsem)
            fire_scatters(g, g % RING)

            @pl.when(g < QSTEPS - RING + 1)
            def _():
                fire_gathers(g + RING - 1, (g + RING - 1) % RING)
            return 0

        ia = pltpu.async_copy(srcr.at[sub].at[pl.ds(0, QSTEPS)], src_all, gsem)
        ib = pltpu.async_copy(tgtr.at[sub].at[pl.ds(0, QSTEPS)], tgt_all, gsem)
        st.wait()
        ia.wait()
        ib.wait()
        plsc.subcore_barrier()
        for p in range(RING - 1):
            fire_gathers(p, p)

        def zbody(z, _):
            pltpu.sync_copy(zer_v,
                            agg_sh.at[pl.ds(sub * 5008 + z * ZROWS, ZROWS)])
            return 0
        lax.fori_loop(0, 39, zbody, 0)
        pltpu.sync_copy(zer_v.at[pl.ds(0, 16)],
                        agg_sh.at[pl.ds(sub * 5008 + 39 * ZROWS, 16)])
        plsc.subcore_barrier()

        for q in range(STEPS // QSTEPS):
            if q > 0:
                qa = pltpu.async_copy(
                    srcr.at[sub].at[pl.ds(q * QSTEPS, QSTEPS)], src_all, gsem)
                qb = pltpu.async_copy(
                    tgtr.at[sub].at[pl.ds(q * QSTEPS, QSTEPS)], tgt_all, gsem)
                qa.wait()
                qb.wait()
                for p in range(RING - 1):
                    fire_gathers(p, p)
            lax.fori_loop(0, QSTEPS, step, 0)
            drain(ssem)
        plsc.subcore_barrier()
        r_s = sub // 2
        n0 = (sub % 2) * OUT_STRIPE
        pltpu.sync_copy(
            agg_sh.at[pl.ds(sub * OUT_STRIPE, OUT_STRIPE)],
            out.at[r_s, pl.ds(n0, OUT_STRIPE), pl.ds(c * CF, CF)])


_agg_call = pl.kernel(
    _agg_body,
    out_type=jax.ShapeDtypeStruct((R, N, CH), jnp.float32),
    mesh=plsc.VectorSubcoreMesh(core_axis_name="c", subcore_axis_name="s"),
    scratch_types=[
        pltpu.VMEM_SHARED((AGG_ROWS, CF), jnp.float32),
        pltpu.VMEM_SHARED((N, CF), jnp.float32),
        pltpu.VMEM((QSTEPS, GROUP, IDXW), jnp.int32),
        pltpu.VMEM((QSTEPS, GROUP, IDXW), jnp.int32),
        pltpu.VMEM((RING * GROUP * IDXW, CF), jnp.float32),
        pltpu.VMEM((ZROWS, CF), jnp.float32),
        pltpu.SemaphoreType.DMA,
        pltpu.SemaphoreType.DMA,
    ],
    compiler_params=pltpu.CompilerParams(use_tc_tiling_on_sc=False),
)


CG = 8


def _cnt_body(tgtr, out, cnt_sh, tgt_idx, ones_v, zer_v, ssem):
    core = lax.axis_index("c")
    sub = lax.axis_index("s")
    _zero_fill(zer_v, ZROWS)

    def fill(i, _):
        ones_v[i, :] = jnp.ones((CF,), jnp.float32)
        return 0
    lax.fori_loop(0, IDXW, fill, 0)

    def zbody(z, _):
        pltpu.sync_copy(zer_v, cnt_sh.at[pl.ds(sub * 5008 + z * ZROWS, ZROWS)])
        return 0
    lax.fori_loop(0, 39, zbody, 0)
    pltpu.sync_copy(zer_v.at[pl.ds(0, 16)],
                    cnt_sh.at[pl.ds(sub * 5008 + 39 * ZROWS, 16)])
    plsc.subcore_barrier()
    half = ROWS // 2
    per_tile = half // TILES

    def step(g, _):
        base = core * half + sub * per_tile + g * CG
        pltpu.sync_copy(tgtr.at[pl.ds(base, CG)], tgt_idx)
        scs = []
        for j in range(CG):
            scs.append(pltpu.async_copy(
                ones_v, cnt_sh.at[tgt_idx.at[j]], ssem, add=True))
        for s in scs:
            s.wait()
        return 0

    lax.fori_loop(0, per_tile // CG, step, 0)
    plsc.subcore_barrier()
    pltpu.sync_copy(cnt_sh.at[pl.ds(sub * OUT_STRIPE, OUT_STRIPE)],
                    out.at[core].at[pl.ds(sub * OUT_STRIPE, OUT_STRIPE)])


_cnt_call = pl.kernel(
    _cnt_body,
    out_type=jax.ShapeDtypeStruct((2, AGG_REAL, CF), jnp.float32),
    mesh=plsc.VectorSubcoreMesh(core_axis_name="c", subcore_axis_name="s"),
    scratch_types=[
        pltpu.VMEM_SHARED((AGG_ROWS, CF), jnp.float32),
        pltpu.VMEM((CG, IDXW), jnp.int32),
        pltpu.VMEM((IDXW, CF), jnp.float32),
        pltpu.VMEM((ZROWS, CF), jnp.float32),
        pltpu.SemaphoreType.DMA,
    ],
    compiler_params=pltpu.CompilerParams(use_tc_tiling_on_sc=False),
)


def _dense_body(relu, x_ref, agg_ref, inv_ref, w_ref, root_ref, b_ref, o_ref):
    acc = jnp.dot(x_ref[...], root_ref[...],
                  preferred_element_type=jnp.float32) + b_ref[...]
    for r in range(R):
        scaled = agg_ref[r] * inv_ref[:, r:r + 1]
        acc = acc + jnp.dot(scaled, w_ref[r], preferred_element_type=jnp.float32)
    if relu:
        acc = jnp.maximum(acc, 0.0)
    o_ref[...] = acc


def _dense_call(x, agg, invT, W, root, b, relu):
    return pl.pallas_call(
        functools.partial(_dense_body, relu),
        grid=(N // BN,),
        in_specs=[
            pl.BlockSpec((BN, CH), lambda i: (i, 0)),
            pl.BlockSpec((R, BN, CH), lambda i: (0, i, 0)),
            pl.BlockSpec((BN, R), lambda i: (i, 0)),
            pl.BlockSpec((R, CH, CH), lambda i: (0, 0, 0)),
            pl.BlockSpec((CH, CH), lambda i: (0, 0)),
            pl.BlockSpec((1, CH), lambda i: (0, 0)),
        ],
        out_specs=pl.BlockSpec((BN, CH), lambda i: (i, 0)),
        out_shape=jax.ShapeDtypeStruct((N, CH), jnp.float32),
    )(x, agg, invT, W, root, b)


def kernel(x, edge_index, edge_type, W1, root1, b1, W2, root2, b2, W3, root3, b3):
    src = edge_index[0]
    dst = edge_index[1]
    tgt = edge_type * N + dst
    pad = E_PAD - E
    srcr = jnp.concatenate([src, jnp.zeros((pad,), jnp.int32)]).reshape(ROWS, IDXW)
    tgtr = jnp.concatenate([tgt, jnp.full((pad,), DUMMY, jnp.int32)]).reshape(ROWS, IDXW)
    srcr4 = srcr.reshape(TILES, STEPS, GROUP, IDXW)
    tgtr4 = tgtr.reshape(TILES, STEPS, GROUP, IDXW)

    cnt2 = _cnt_call(tgtr)
    cnt = cnt2[0, :, 0] + cnt2[1, :, 0]
    invT = (1.0 / jnp.maximum(cnt, 1.0)).reshape(R, N).T

    def layer(h, W, root, b, relu):
        agg = _agg_call(h, srcr4, tgtr4)
        return _dense_call(h, agg, invT, W, root, b.reshape(1, CH), relu)

    h1 = layer(x, W1, root1, b1, True)
    h2 = layer(h1, W2, root2, b2, True)
    return layer(h2, W3, root3, b3, False)
